# baseline (speedup 1.0000x reference)
import functools

import jax
import jax.numpy as jnp
from jax import lax
from jax.experimental import pallas as pl
from jax.experimental.pallas import tpu as pltpu
from jax.experimental.pallas import tpu_sc as plsc

_LANES = 16


@functools.partial(jax.jit, static_argnames=("nw",))
def _sc_embed(x, token_table, pos_table, *, nw):
    batch, seq = x.shape
    hidden = token_table.shape[1]
    spw = seq // nw
    win = 128

    mesh = plsc.VectorSubcoreMesh(core_axis_name="c", subcore_axis_name="s")

    @functools.partial(
        pl.kernel,
        out_type=jax.ShapeDtypeStruct((batch * seq, hidden), jnp.float32),
        mesh=mesh,
        scratch_types=[
            pltpu.VMEM((batch, win), jnp.int32),
            pltpu.VMEM((batch * spw, hidden), jnp.float32),
            [pltpu.SemaphoreType.DMA] * 4,
            [pltpu.SemaphoreType.DMA] * 4,
            [pltpu.SemaphoreType.DMA] * 4,
            pltpu.SemaphoreType.DMA,
        ],
    )
    def body(x_hbm, tok_hbm, pos_hbm, out_hbm, idx_v, rows_v,
             isems, psems, gsems, wsem):
        wid = lax.axis_index("s") * 2 + lax.axis_index("c")
        s0 = wid * spw
        w0 = (s0 // win) * win
        off = s0 - w0

        icps = [
            pltpu.async_copy(
                x_hbm.at[pl.ds(b, 1), pl.ds(w0, win)],
                idx_v.at[pl.ds(b, 1)],
                isems[b],
            )
            for b in range(batch)
        ]
        pcps = [
            pltpu.async_copy(
                pos_hbm.at[pl.ds(s0, spw)],
                rows_v.at[pl.ds(b * spw, spw)],
                psems[b],
            )
            for b in range(batch)
        ]

        gcps = []
        for b in range(batch):
            icps[b].wait()
            pcps[b].wait()
            gcps.append(
                pltpu.async_copy(
                    tok_hbm.at[idx_v.at[b, pl.ds(off, spw)]],
                    rows_v.at[pl.ds(b * spw, spw)],
                    gsems[b],
                    add=True,
                )
            )

        wcps = []
        for b in range(batch):
            gcps[b].wait()
            wcps.append(
                pltpu.async_copy(
                    rows_v.at[pl.ds(b * spw, spw)],
                    out_hbm.at[pl.ds(b * seq + s0, spw)],
                    wsem,
                )
            )
        for cp in wcps:
            cp.wait()

    return body(x, token_table, pos_table)


def kernel(x, token_table, pos_table):
    batch, seq = x.shape
    hidden = token_table.shape[1]
    out = _sc_embed(x.astype(jnp.int32), token_table, pos_table, nw=32)
    return out.reshape(batch, seq, hidden)


# speedup vs baseline: 1.0301x; 1.0301x over previous
import functools

import jax
import jax.numpy as jnp
from jax import lax
from jax.experimental import pallas as pl
from jax.experimental.pallas import tpu as pltpu
from jax.experimental.pallas import tpu_sc as plsc

_LANES = 16


@functools.partial(jax.jit, static_argnames=("nw",))
def _sc_embed(x, token_table, pos_table, *, nw):
    batch, seq = x.shape
    hidden = token_table.shape[1]
    spw = seq // nw
    half = spw // 2
    win = 128
    lanes = hidden // _LANES

    mesh = plsc.VectorSubcoreMesh(core_axis_name="c", subcore_axis_name="s")

    @functools.partial(
        pl.kernel,
        out_type=jax.ShapeDtypeStruct((batch * seq, hidden), jnp.float32),
        mesh=mesh,
        scratch_types=[
            pltpu.VMEM((batch, win), jnp.int32),
            pltpu.VMEM((batch * spw, hidden), jnp.float32),
            pltpu.VMEM((spw, hidden), jnp.float32),
            [pltpu.SemaphoreType.DMA] * 4,
            [pltpu.SemaphoreType.DMA] * 8,
            pltpu.SemaphoreType.DMA,
            pltpu.SemaphoreType.DMA,
        ],
    )
    def body(x_hbm, tok_hbm, pos_hbm, out_hbm, idx_v, rows_v, pos_v,
             isems, gsems, psem, wsem):
        wid = lax.axis_index("s") * 2 + lax.axis_index("c")
        s0 = wid * spw
        w0 = (s0 // win) * win
        off = s0 - w0

        icps = [
            pltpu.async_copy(
                x_hbm.at[pl.ds(b, 1), pl.ds(w0, win)],
                idx_v.at[pl.ds(b, 1)],
                isems[b],
            )
            for b in range(batch)
        ]
        pcp = pltpu.async_copy(pos_hbm.at[pl.ds(s0, spw)], pos_v, psem)
        pcp.wait()

        gcps = []
        for h in range(2):
            def prefill_row(r, carry, _h=h):
                base = _h * half + r
                for j in range(lanes):
                    sl = pl.ds(j * _LANES, _LANES)
                    v = pos_v[base, sl]
                    for b in range(batch):
                        rows_v[b * spw + base, sl] = v
                return carry

            lax.fori_loop(0, half, prefill_row, 0)
            for b in range(batch):
                if h == 0:
                    icps[b].wait()
                gcps.append(
                    pltpu.async_copy(
                        tok_hbm.at[idx_v.at[b, pl.ds(off + h * half, half)]],
                        rows_v.at[pl.ds(b * spw + h * half, half)],
                        gsems[h * batch + b],
                        add=True,
                    )
                )

        wcps = []
        for c, cp in enumerate(gcps):
            h, b = c // batch, c % batch
            cp.wait()
            wcps.append(
                pltpu.async_copy(
                    rows_v.at[pl.ds(b * spw + h * half, half)],
                    out_hbm.at[pl.ds(b * seq + s0 + h * half, half)],
                    wsem,
                )
            )
        for cp in wcps:
            cp.wait()

    return body(x, token_table, pos_table)


def kernel(x, token_table, pos_table):
    batch, seq = x.shape
    hidden = token_table.shape[1]
    out = _sc_embed(x.astype(jnp.int32), token_table, pos_table, nw=32)
    return out.reshape(batch, seq, hidden)


# speedup vs baseline: 1.0464x; 1.0159x over previous
import functools

import jax
import jax.numpy as jnp
from jax import lax
from jax.experimental import pallas as pl
from jax.experimental.pallas import tpu as pltpu
from jax.experimental.pallas import tpu_sc as plsc

_LANES = 16


@functools.partial(jax.jit, static_argnames=("nw",))
def _sc_embed(x, token_table, pos_table, *, nw):
    batch, seq = x.shape
    hidden = token_table.shape[1]
    spw = seq // nw
    half = spw // 2
    win = 128
    lanes = hidden // _LANES

    mesh = plsc.VectorSubcoreMesh(core_axis_name="c", subcore_axis_name="s")

    @functools.partial(
        pl.kernel,
        out_type=jax.ShapeDtypeStruct((batch * seq, hidden), jnp.float32),
        mesh=mesh,
        scratch_types=[
            pltpu.VMEM((batch, win), jnp.int32),
            pltpu.VMEM((batch * spw, hidden), jnp.float32),
            pltpu.VMEM((spw, hidden), jnp.float32),
            [pltpu.SemaphoreType.DMA] * 4,
            [pltpu.SemaphoreType.DMA] * 8,
            pltpu.SemaphoreType.DMA,
            pltpu.SemaphoreType.DMA,
        ],
    )
    def body(x_hbm, tok_hbm, pos_hbm, out_hbm, idx_v, rows_v, pos_v,
             isems, gsems, psem, wsem):
        wid = lax.axis_index("s") * 2 + lax.axis_index("c")
        s0 = wid * spw
        w0 = (s0 // win) * win
        off = s0 - w0

        icps = [
            pltpu.async_copy(
                x_hbm.at[pl.ds(b, 1), pl.ds(w0, win)],
                idx_v.at[pl.ds(b, 1)],
                isems[b],
            )
            for b in range(batch)
        ]
        pcp = pltpu.async_copy(pos_hbm.at[pl.ds(s0, spw)], pos_v, psem)

        gcps = {}
        for b in range(batch):
            icps[b].wait()
            gcps[(0, b)] = pltpu.async_copy(
                tok_hbm.at[idx_v.at[b, pl.ds(off, half)]],
                rows_v.at[pl.ds(b * spw, half)],
                gsems[b],
            )
        for b in range(batch):
            gcps[(1, b)] = pltpu.async_copy(
                tok_hbm.at[idx_v.at[b, pl.ds(off + half, half)]],
                rows_v.at[pl.ds(b * spw + half, half)],
                gsems[batch + b],
            )
        pcp.wait()

        wcps = []
        for h in range(2):
            for b in range(batch):
                gcps[(h, b)].wait()

            def add_row(r, carry, _h=h):
                base = _h * half + r
                for j in range(lanes):
                    sl = pl.ds(j * _LANES, _LANES)
                    v = pos_v[base, sl]
                    for b in range(batch):
                        plsc.addupdate(rows_v.at[b * spw + base, sl], v)
                return carry

            lax.fori_loop(0, half, add_row, 0)
            for b in range(batch):
                wcps.append(
                    pltpu.async_copy(
                        rows_v.at[pl.ds(b * spw + h * half, half)],
                        out_hbm.at[pl.ds(b * seq + s0 + h * half, half)],
                        wsem,
                    )
                )
        for cp in wcps:
            cp.wait()

    return body(x, token_table, pos_table)


def kernel(x, token_table, pos_table):
    batch, seq = x.shape
    hidden = token_table.shape[1]
    out = _sc_embed(x.astype(jnp.int32), token_table, pos_table, nw=32)
    return out.reshape(batch, seq, hidden)
